# initial kernel scaffold (unmeasured)
import jax
import jax.numpy as jnp
from jax import lax
from jax.experimental import pallas as pl
from jax.experimental.pallas import tpu as pltpu

N_DEV = 8
M = 1024
N = 1024
N_STAGES = 3

PARTS = [(0, 1024, (0, 1, 2))]


def _coords(my):
    q = my % 4
    cx = jnp.where((q == 1) | (q == 2), 1, 0).astype(jnp.int32)
    cy = q // 2
    cz = my // 4
    return (cx, cy, cz)


def _partner(my, axis):
    q = my % 4
    if axis == 0:
        return my + 1 - 2 * (q % 2)
    if axis == 1:
        return my - 2 * q + 3
    return (my + 4) % N_DEV


def kernel(x, w_mat):
    def body(x_ref, w_ref, out_ref, *scratch):
        bufs = scratch[:-2]
        send_sems, recv_sems = scratch[-2:]

        def buf(p, step, kind):
            return bufs[(p * 6 + step) * 2 + kind]

        my = lax.axis_index("i")
        coords = _coords(my)

        out_ref[:, :] = jnp.dot(
            x_ref[:, :].astype(jnp.bfloat16),
            w_ref[:, :].astype(jnp.bfloat16),
            preferred_element_type=jnp.float32,
        )

        bases = [jnp.int32(0) for _ in PARTS]

        for s in range(N_STAGES):
            L = M >> (s + 1)
            rdmas = []
            for p, (c0, nc, order) in enumerate(PARTS):
                axis = order[s]
                c = coords[axis]
                keep = bases[p] + c * L
                send = bases[p] + (1 - c) * L
                sb = buf(p, s, 0)
                sb[:, :] = out_ref[pl.ds(send, L), pl.ds(c0, nc)].astype(
                    jnp.bfloat16
                )
                rdma = pltpu.make_async_remote_copy(
                    src_ref=sb,
                    dst_ref=buf(p, s, 1),
                    send_sem=send_sems.at[p * 6 + s],
                    recv_sem=recv_sems.at[p * 6 + s],
                    device_id=_partner(my, axis),
                    device_id_type=pl.DeviceIdType.LOGICAL,
                )
                rdma.start()
                rdmas.append((rdma, keep))
            for p, (c0, nc, order) in enumerate(PARTS):
                rdma, keep = rdmas[p]
                rdma.wait()
                rb = buf(p, s, 1)
                out_ref[pl.ds(keep, L), pl.ds(c0, nc)] = (
                    out_ref[pl.ds(keep, L), pl.ds(c0, nc)]
                    + rb[:, :].astype(jnp.float32)
                )
                bases[p] = keep

        for s in range(N_STAGES):
            L = M >> (N_STAGES - s)
            rdmas = []
            for p, (c0, nc, order) in enumerate(PARTS):
                axis = order[N_STAGES - 1 - s]
                c = coords[axis]
                sb = buf(p, 3 + s, 0)
                sb[:, :] = out_ref[pl.ds(bases[p], L), pl.ds(c0, nc)].astype(
                    jnp.bfloat16
                )
                rdma = pltpu.make_async_remote_copy(
                    src_ref=sb,
                    dst_ref=buf(p, 3 + s, 1),
                    send_sem=send_sems.at[p * 6 + 3 + s],
                    recv_sem=recv_sems.at[p * 6 + 3 + s],
                    device_id=_partner(my, axis),
                    device_id_type=pl.DeviceIdType.LOGICAL,
                )
                rdma.start()
                parent = bases[p] - c * L
                rdmas.append((rdma, parent + (1 - c) * L, parent))
            for p, (c0, nc, order) in enumerate(PARTS):
                rdma, pbase, parent = rdmas[p]
                rdma.wait()
                rb = buf(p, 3 + s, 1)
                out_ref[pl.ds(pbase, L), pl.ds(c0, nc)] = rb[:, :].astype(
                    jnp.float32
                )
                bases[p] = parent

    scratch_shapes = []
    for (c0, nc, order) in PARTS:
        for s in range(N_STAGES):
            L = M >> (s + 1)
            scratch_shapes += [pltpu.VMEM((L, nc), jnp.bfloat16)] * 2
        for s in range(N_STAGES):
            L = M >> (N_STAGES - s)
            scratch_shapes += [pltpu.VMEM((L, nc), jnp.bfloat16)] * 2
    n_sems = len(PARTS) * 6
    scratch_shapes += [
        pltpu.SemaphoreType.DMA((n_sems,)),
        pltpu.SemaphoreType.DMA((n_sems,)),
    ]

    return pl.pallas_call(
        body,
        out_shape=jax.ShapeDtypeStruct((M, N), jnp.float32),
        in_specs=[
            pl.BlockSpec(memory_space=pltpu.VMEM),
            pl.BlockSpec(memory_space=pltpu.VMEM),
        ],
        out_specs=pl.BlockSpec(memory_space=pltpu.VMEM),
        scratch_shapes=scratch_shapes,
        compiler_params=pltpu.CompilerParams(collective_id=0),
    )(x, w_mat)


# baseline (device time: 61868 ns/iter reference)
import jax
import jax.numpy as jnp
from jax import lax
from jax.experimental import pallas as pl
from jax.experimental.pallas import tpu as pltpu

N_DEV = 8
M = 1024
N = 1024
N_STAGES = 3

PARTS = [(0, 1024, (0, 1, 2))]


def _coords(my):
    q = my % 4
    cx = jnp.where((q == 1) | (q == 2), 1, 0).astype(jnp.int32)
    cy = q // 2
    cz = my // 4
    return (cx, cy, cz)


def _partner(my, axis):
    q = my % 4
    if axis == 0:
        return my + 1 - 2 * (q % 2)
    if axis == 1:
        return my - 2 * q + 3
    return (my + 4) % N_DEV


def kernel(x, w_mat):
    def body(x_ref, w_ref, out_ref, *scratch):
        bufs = scratch[:-2]
        send_sems, recv_sems = scratch[-2:]

        def buf(p, step, kind):
            return bufs[(p * 6 + step) * 2 + kind]

        my = lax.axis_index("i")
        coords = _coords(my)

        out_ref[:, :] = jnp.dot(
            x_ref[:, :].astype(jnp.bfloat16),
            w_ref[:, :].astype(jnp.bfloat16),
            preferred_element_type=jnp.float32,
        )

        bases = [jnp.int32(0) for _ in PARTS]

        for s in range(N_STAGES):
            L = M >> (s + 1)
            rdmas = []
            for p, (c0, nc, order) in enumerate(PARTS):
                axis = order[s]
                c = coords[axis]
                keep = bases[p] + c * L
                send = bases[p] + (1 - c) * L
                sb = buf(p, s, 0)
                sb[:, :] = out_ref[pl.ds(send, L), pl.ds(c0, nc)].astype(
                    jnp.bfloat16
                )
                rdma = pltpu.make_async_remote_copy(
                    src_ref=sb,
                    dst_ref=buf(p, s, 1),
                    send_sem=send_sems.at[p * 6 + s],
                    recv_sem=recv_sems.at[p * 6 + s],
                    device_id=_partner(my, axis),
                    device_id_type=pl.DeviceIdType.LOGICAL,
                )
                rdma.start()
                rdmas.append((rdma, keep))
            for p, (c0, nc, order) in enumerate(PARTS):
                rdma, keep = rdmas[p]
                rdma.wait()
                rb = buf(p, s, 1)
                out_ref[pl.ds(keep, L), pl.ds(c0, nc)] = (
                    out_ref[pl.ds(keep, L), pl.ds(c0, nc)]
                    + rb[:, :].astype(jnp.float32)
                )
                bases[p] = keep

        for s in range(N_STAGES):
            L = M >> (N_STAGES - s)
            rdmas = []
            for p, (c0, nc, order) in enumerate(PARTS):
                axis = order[N_STAGES - 1 - s]
                c = coords[axis]
                sb = buf(p, 3 + s, 0)
                sb[:, :] = out_ref[pl.ds(bases[p], L), pl.ds(c0, nc)].astype(
                    jnp.bfloat16
                )
                rdma = pltpu.make_async_remote_copy(
                    src_ref=sb,
                    dst_ref=buf(p, 3 + s, 1),
                    send_sem=send_sems.at[p * 6 + 3 + s],
                    recv_sem=recv_sems.at[p * 6 + 3 + s],
                    device_id=_partner(my, axis),
                    device_id_type=pl.DeviceIdType.LOGICAL,
                )
                rdma.start()
                parent = bases[p] - c * L
                rdmas.append((rdma, parent + (1 - c) * L, parent))
            for p, (c0, nc, order) in enumerate(PARTS):
                rdma, pbase, parent = rdmas[p]
                rdma.wait()
                rb = buf(p, 3 + s, 1)
                out_ref[pl.ds(pbase, L), pl.ds(c0, nc)] = rb[:, :].astype(
                    jnp.float32
                )
                bases[p] = parent

    scratch_shapes = []
    for (c0, nc, order) in PARTS:
        for s in range(N_STAGES):
            L = M >> (s + 1)
            scratch_shapes += [pltpu.VMEM((L, nc), jnp.bfloat16)] * 2
        for s in range(N_STAGES):
            L = M >> (N_STAGES - s)
            scratch_shapes += [pltpu.VMEM((L, nc), jnp.bfloat16)] * 2
    n_sems = len(PARTS) * 6
    scratch_shapes += [
        pltpu.SemaphoreType.DMA((n_sems,)),
        pltpu.SemaphoreType.DMA((n_sems,)),
    ]

    return pl.pallas_call(
        body,
        out_shape=jax.ShapeDtypeStruct((M, N), jnp.float32),
        in_specs=[
            pl.BlockSpec(memory_space=pltpu.VMEM),
            pl.BlockSpec(memory_space=pltpu.VMEM),
        ],
        out_specs=pl.BlockSpec(memory_space=pltpu.VMEM),
        scratch_shapes=scratch_shapes,
    )(x, w_mat)


# device time: 38149 ns/iter; 1.6217x vs baseline; 1.6217x over previous
import jax
import jax.numpy as jnp
from jax import lax
from jax.experimental import pallas as pl
from jax.experimental.pallas import tpu as pltpu

N_DEV = 8
M = 1024
N = 1024
N_STAGES = 3

PARTS = [
    (0, 384, (0, 1, 2)),
    (384, 384, (1, 2, 0)),
    (768, 256, (2, 0, 1)),
]


def _coords(my):
    q = my % 4
    cx = jnp.where((q == 1) | (q == 2), 1, 0).astype(jnp.int32)
    cy = q // 2
    cz = my // 4
    return (cx, cy, cz)


def _partner(my, axis):
    q = my % 4
    if axis == 0:
        return my + 1 - 2 * (q % 2)
    if axis == 1:
        return my - 2 * q + 3
    return (my + 4) % N_DEV


def kernel(x, w_mat):
    def body(x_ref, w_ref, out_ref, *scratch):
        bufs = scratch[:-2]
        send_sems, recv_sems = scratch[-2:]

        def buf(p, step, kind):
            return bufs[(p * 6 + step) * 2 + kind]

        my = lax.axis_index("i")
        coords = _coords(my)

        out_ref[:, :] = jnp.dot(
            x_ref[:, :].astype(jnp.bfloat16),
            w_ref[:, :].astype(jnp.bfloat16),
            preferred_element_type=jnp.float32,
        )

        bases = [jnp.int32(0) for _ in PARTS]

        for s in range(N_STAGES):
            L = M >> (s + 1)
            rdmas = []
            for p, (c0, nc, order) in enumerate(PARTS):
                axis = order[s]
                c = coords[axis]
                keep = bases[p] + c * L
                send = bases[p] + (1 - c) * L
                sb = buf(p, s, 0)
                sb[:, :] = out_ref[pl.ds(send, L), pl.ds(c0, nc)].astype(
                    jnp.bfloat16
                )
                rdma = pltpu.make_async_remote_copy(
                    src_ref=sb,
                    dst_ref=buf(p, s, 1),
                    send_sem=send_sems.at[p * 6 + s],
                    recv_sem=recv_sems.at[p * 6 + s],
                    device_id=_partner(my, axis),
                    device_id_type=pl.DeviceIdType.LOGICAL,
                )
                rdma.start()
                rdmas.append((rdma, keep))
            for p, (c0, nc, order) in enumerate(PARTS):
                rdma, keep = rdmas[p]
                rdma.wait()
                rb = buf(p, s, 1)
                out_ref[pl.ds(keep, L), pl.ds(c0, nc)] = (
                    out_ref[pl.ds(keep, L), pl.ds(c0, nc)]
                    + rb[:, :].astype(jnp.float32)
                )
                bases[p] = keep

        for s in range(N_STAGES):
            L = M >> (N_STAGES - s)
            rdmas = []
            for p, (c0, nc, order) in enumerate(PARTS):
                axis = order[N_STAGES - 1 - s]
                c = coords[axis]
                sb = buf(p, 3 + s, 0)
                sb[:, :] = out_ref[pl.ds(bases[p], L), pl.ds(c0, nc)].astype(
                    jnp.bfloat16
                )
                rdma = pltpu.make_async_remote_copy(
                    src_ref=sb,
                    dst_ref=buf(p, 3 + s, 1),
                    send_sem=send_sems.at[p * 6 + 3 + s],
                    recv_sem=recv_sems.at[p * 6 + 3 + s],
                    device_id=_partner(my, axis),
                    device_id_type=pl.DeviceIdType.LOGICAL,
                )
                rdma.start()
                parent = bases[p] - c * L
                rdmas.append((rdma, parent + (1 - c) * L, parent))
            for p, (c0, nc, order) in enumerate(PARTS):
                rdma, pbase, parent = rdmas[p]
                rdma.wait()
                rb = buf(p, 3 + s, 1)
                out_ref[pl.ds(pbase, L), pl.ds(c0, nc)] = rb[:, :].astype(
                    jnp.float32
                )
                bases[p] = parent

    scratch_shapes = []
    for (c0, nc, order) in PARTS:
        for s in range(N_STAGES):
            L = M >> (s + 1)
            scratch_shapes += [pltpu.VMEM((L, nc), jnp.bfloat16)] * 2
        for s in range(N_STAGES):
            L = M >> (N_STAGES - s)
            scratch_shapes += [pltpu.VMEM((L, nc), jnp.bfloat16)] * 2
    n_sems = len(PARTS) * 6
    scratch_shapes += [
        pltpu.SemaphoreType.DMA((n_sems,)),
        pltpu.SemaphoreType.DMA((n_sems,)),
    ]

    return pl.pallas_call(
        body,
        out_shape=jax.ShapeDtypeStruct((M, N), jnp.float32),
        in_specs=[
            pl.BlockSpec(memory_space=pltpu.VMEM),
            pl.BlockSpec(memory_space=pltpu.VMEM),
        ],
        out_specs=pl.BlockSpec(memory_space=pltpu.VMEM),
        scratch_shapes=scratch_shapes,
    )(x, w_mat)


# device time: 34253 ns/iter; 1.8062x vs baseline; 1.1137x over previous
import jax
import jax.numpy as jnp
from jax import lax
from jax.experimental import pallas as pl
from jax.experimental.pallas import tpu as pltpu

N_DEV = 8
M = 1024
N = 1024
N_STAGES = 3

PARTS = [
    (0, 384, (0, 1, 2)),
    (384, 384, (1, 2, 0)),
    (768, 256, (2, 0, 1)),
]


def _coords(my):
    q = my % 4
    cx = jnp.where((q == 1) | (q == 2), 1, 0).astype(jnp.int32)
    cy = q // 2
    cz = my // 4
    return (cx, cy, cz)


def _partner(my, axis):
    q = my % 4
    if axis == 0:
        return my + 1 - 2 * (q % 2)
    if axis == 1:
        return my - 2 * q + 3
    return (my + 4) % N_DEV


def kernel(x, w_mat):
    n_parts = len(PARTS)

    def body(x_ref, w_ref, out_ref, *scratch):
        bufs = scratch[: 7 * n_parts]
        send_sems, recv_sems = scratch[7 * n_parts:]

        def sb(p, s):
            return bufs[7 * p + 2 * s]

        def rb(p, s):
            return bufs[7 * p + 2 * s + 1]

        def shadow(p):
            return bufs[7 * p + 6]

        my = lax.axis_index("i")
        coords = _coords(my)
        all_rdmas = []

        def exchange(p, step, src, dst, axis):
            rdma = pltpu.make_async_remote_copy(
                src_ref=src,
                dst_ref=dst,
                send_sem=send_sems.at[6 * p + step],
                recv_sem=recv_sems.at[6 * p + step],
                device_id=_partner(my, axis),
                device_id_type=pl.DeviceIdType.LOGICAL,
            )
            all_rdmas.append(rdma)
            return rdma

        out_ref[:, :] = jnp.dot(
            x_ref[:, :].astype(jnp.bfloat16),
            w_ref[:, :].astype(jnp.bfloat16),
            preferred_element_type=jnp.float32,
        )

        barrier = pltpu.get_barrier_semaphore()
        for axis in range(3):
            pl.semaphore_signal(
                barrier,
                inc=1,
                device_id=_partner(my, axis),
                device_id_type=pl.DeviceIdType.LOGICAL,
            )
        pl.semaphore_wait(barrier, 3)

        bases = [jnp.int32(0) for _ in PARTS]
        rdmas = []
        for p, (c0, nc, order) in enumerate(PARTS):
            c = coords[order[0]]
            L = M // 2
            send = bases[p] + (1 - c) * L
            sb(p, 0)[:, :] = out_ref[pl.ds(send, L), pl.ds(c0, nc)].astype(
                jnp.bfloat16
            )
            r = exchange(p, 0, sb(p, 0), rb(p, 0), order[0])
            r.start()
            rdmas.append(r)

        for s in range(N_STAGES):
            L = M >> (s + 1)
            L2 = L // 2
            next_rdmas = []
            for p, (c0, nc, order) in enumerate(PARTS):
                c = coords[order[s]]
                keep = bases[p] + c * L
                rdmas[p].wait_recv()
                cols = pl.ds(c0, nc)
                if s < N_STAGES - 1:
                    cn = coords[order[s + 1]]
                    rel_send = (1 - cn) * L2
                    rel_keep = cn * L2
                    h_send = out_ref[pl.ds(keep + rel_send, L2), cols] + rb(
                        p, s
                    )[pl.ds(rel_send, L2), :].astype(jnp.float32)
                    sb(p, s + 1)[:, :] = h_send.astype(jnp.bfloat16)
                    r = exchange(
                        p, s + 1, sb(p, s + 1), rb(p, s + 1), order[s + 1]
                    )
                    r.start()
                    next_rdmas.append(r)
                    out_ref[pl.ds(keep + rel_send, L2), cols] = h_send
                    out_ref[pl.ds(keep + rel_keep, L2), cols] = out_ref[
                        pl.ds(keep + rel_keep, L2), cols
                    ] + rb(p, s)[pl.ds(rel_keep, L2), :].astype(jnp.float32)
                else:
                    full = out_ref[pl.ds(keep, L), cols] + rb(p, s)[
                        :, :
                    ].astype(jnp.float32)
                    shadow(p)[pl.ds(keep, L), :] = full.astype(jnp.bfloat16)
                    r = exchange(
                        p,
                        3,
                        shadow(p).at[pl.ds(keep, L), :],
                        shadow(p).at[pl.ds(keep, L), :],
                        order[N_STAGES - 1],
                    )
                    r.start()
                    next_rdmas.append(r)
                    out_ref[pl.ds(keep, L), cols] = full
                bases[p] = keep
            rdmas = next_rdmas

        for t in range(N_STAGES):
            L = M >> (N_STAGES - t)
            next_rdmas = []
            for p, (c0, nc, order) in enumerate(PARTS):
                axis = order[N_STAGES - 1 - t]
                c = coords[axis]
                parent = bases[p] - c * L
                pbase = parent + (1 - c) * L
                rdmas[p].wait_recv()
                if t < N_STAGES - 1:
                    r = exchange(
                        p,
                        3 + t + 1,
                        shadow(p).at[pl.ds(parent, 2 * L), :],
                        shadow(p).at[pl.ds(parent, 2 * L), :],
                        order[N_STAGES - 2 - t],
                    )
                    r.start()
                    next_rdmas.append(r)
                out_ref[pl.ds(pbase, L), pl.ds(c0, nc)] = shadow(p)[
                    pl.ds(pbase, L), :
                ].astype(jnp.float32)
                bases[p] = parent
            rdmas = next_rdmas

        for r in all_rdmas:
            r.wait_send()

    scratch_shapes = []
    for (c0, nc, order) in PARTS:
        for s in range(N_STAGES):
            L = M >> (s + 1)
            scratch_shapes += [pltpu.VMEM((L, nc), jnp.bfloat16)] * 2
        scratch_shapes.append(pltpu.VMEM((M, nc), jnp.bfloat16))
    n_sems = n_parts * 6
    scratch_shapes += [
        pltpu.SemaphoreType.DMA((n_sems,)),
        pltpu.SemaphoreType.DMA((n_sems,)),
    ]

    return pl.pallas_call(
        body,
        out_shape=jax.ShapeDtypeStruct((M, N), jnp.float32),
        in_specs=[
            pl.BlockSpec(memory_space=pltpu.VMEM),
            pl.BlockSpec(memory_space=pltpu.VMEM),
        ],
        out_specs=pl.BlockSpec(memory_space=pltpu.VMEM),
        scratch_shapes=scratch_shapes,
        compiler_params=pltpu.CompilerParams(collective_id=0),
    )(x, w_mat)
